# baseline (speedup 1.0000x reference)
import functools

import jax
import jax.numpy as jnp
from jax import lax
from jax.experimental import pallas as pl
from jax.experimental.pallas import tpu as pltpu
from jax.experimental.pallas import tpu_sc as plsc


def kernel(table_0, table_1, table_2, indices_0, indices_1, indices_2):
    B = indices_0.shape[0]
    info = plsc.get_sparse_core_info()
    NC, NS = info.num_cores, info.num_subcores
    NW = NC * NS
    assert B % (8 * NW) == 0
    bpw = B // NW
    dims = (table_0.shape[1], table_1.shape[1], table_2.shape[1])
    mesh = plsc.VectorSubcoreMesh(core_axis_name="c", subcore_axis_name="s")

    @functools.partial(
        pl.kernel,
        mesh=mesh,
        compiler_params=pltpu.CompilerParams(use_tc_tiling_on_sc=False),
        out_type=tuple(
            jax.ShapeDtypeStruct((B, d), jnp.float32) for d in dims
        ),
        scratch_types=[
            pltpu.VMEM((bpw,), jnp.int32),
            pltpu.VMEM((bpw,), jnp.int32),
            pltpu.VMEM((bpw,), jnp.int32),
            pltpu.VMEM((bpw, dims[0]), jnp.float32),
            pltpu.VMEM((bpw, dims[1]), jnp.float32),
            pltpu.VMEM((bpw, dims[2]), jnp.float32),
            pltpu.SemaphoreType.DMA,
            pltpu.SemaphoreType.DMA,
            pltpu.SemaphoreType.DMA,
        ],
    )
    def gather3(t0, t1, t2, i0, i1, i2, o0, o1, o2,
                iv0, iv1, iv2, r0, r1, r2, s0, s1, s2):
        wid = lax.axis_index("s") * NC + lax.axis_index("c")
        base = wid * bpw
        pltpu.sync_copy(i0.at[pl.ds(base, bpw)], iv0)
        pltpu.sync_copy(i1.at[pl.ds(base, bpw)], iv1)
        pltpu.sync_copy(i2.at[pl.ds(base, bpw)], iv2)
        c0 = pltpu.async_copy(t0.at[iv0], r0, s0)
        c1 = pltpu.async_copy(t1.at[iv1], r1, s1)
        c2 = pltpu.async_copy(t2.at[iv2], r2, s2)
        c0.wait()
        pltpu.sync_copy(r0, o0.at[pl.ds(base, bpw)])
        c1.wait()
        pltpu.sync_copy(r1, o1.at[pl.ds(base, bpw)])
        c2.wait()
        pltpu.sync_copy(r2, o2.at[pl.ds(base, bpw)])

    return gather3(table_0, table_1, table_2, indices_0, indices_1, indices_2)


# speedup vs baseline: 2.3617x; 2.3617x over previous
import functools

import jax
import jax.numpy as jnp
from jax import lax
from jax.experimental import pallas as pl
from jax.experimental.pallas import tpu as pltpu
from jax.experimental.pallas import tpu_sc as plsc

_CH = 128


def kernel(table_0, table_1, table_2, indices_0, indices_1, indices_2):
    B = indices_0.shape[0]
    V = table_0.shape[0]
    info = plsc.get_sparse_core_info()
    NC, NS = info.num_cores, info.num_subcores
    NW = NC * NS
    assert B % (8 * NW) == 0 and V % 8 == 0
    bpw = B // NW
    dims = (table_0.shape[1], table_1.shape[1], table_2.shape[1])
    mesh = plsc.VectorSubcoreMesh(core_axis_name="c", subcore_axis_name="s")

    t3 = [t.reshape(V // 8, 8, d)
          for t, d in zip((table_0, table_1, table_2), dims)]

    scratch = []
    for d in dims:
        scratch += [
            pltpu.VMEM((bpw,), jnp.int32),
            pltpu.VMEM((_CH // 8, 8, d), jnp.float32),
            pltpu.SemaphoreType.DMA,
        ]

    @functools.partial(
        pl.kernel,
        mesh=mesh,
        out_type=tuple(
            jax.ShapeDtypeStruct((B // 8, 8, d), jnp.float32) for d in dims
        ),
        scratch_types=scratch,
    )
    def gather3(t0, t1, t2, i0, i1, i2, o0, o1, o2, *scr):
        wid = lax.axis_index("s") * NC + lax.axis_index("c")
        base = wid * bpw

        for n, (t, i, o, d) in enumerate(
            zip((t0, t1, t2), (i0, i1, i2), (o0, o1, o2), dims)
        ):
            iv, rows, sem = scr[3 * n], scr[3 * n + 1], scr[3 * n + 2]
            pltpu.sync_copy(i.at[pl.ds(base, bpw)], iv)

            def chunk(k, _, t=t, o=o, d=d, iv=iv, rows=rows, sem=sem):
                off = k * _CH
                for g in range(_CH // 16):
                    rvec = iv[pl.ds(off + g * 16, 16)]
                    for u in range(16):
                        j = g * 16 + u
                        r = rvec[u]
                        pltpu.async_copy(
                            t.at[pl.ds(r >> 3, 1), pl.ds(r & 7, 1)],
                            rows.at[pl.ds(j >> 3, 1), pl.ds(j & 7, 1)],
                            sem,
                        )
                pltpu.make_async_copy(t.at[pl.ds(0, _CH // 8)], rows, sem).wait()
                pltpu.sync_copy(
                    rows, o.at[pl.ds((base + off) // 8, _CH // 8)])
                return _

            lax.fori_loop(0, bpw // _CH, chunk, 0)

    o0, o1, o2 = gather3(*t3, indices_0, indices_1, indices_2)
    return tuple(o.reshape(B, d) for o, d in zip((o0, o1, o2), dims))


# speedup vs baseline: 2.8772x; 1.2183x over previous
import functools

import jax
import jax.numpy as jnp
from jax import lax
from jax.experimental import pallas as pl
from jax.experimental.pallas import tpu as pltpu
from jax.experimental.pallas import tpu_sc as plsc

_W = 512
_CAP = 4096
_RCAP = 80
_L = 16


def kernel(table_0, table_1, table_2, indices_0, indices_1, indices_2):
    B = indices_0.shape[0]
    V = table_0.shape[0]
    info = plsc.get_sparse_core_info()
    NC, NS = info.num_cores, info.num_subcores
    NW = NC * NS
    dims = (table_0.shape[1], table_1.shape[1], table_2.shape[1])
    padv = ((V + 127) // 128) * 128
    nwin = (V + _W - 1) // _W
    mesh = plsc.VectorSubcoreMesh(core_axis_name="c", subcore_axis_name="s")

    tT3 = [t.T.reshape(d // 8, 8, V)
           for t, d in zip((table_0, table_1, table_2), dims)]

    scratch = [
        pltpu.VMEM((8, 8, _W), jnp.float32),
        pltpu.VMEM((8, 8, _W), jnp.float32),
        pltpu.VMEM((_CAP,), jnp.int32),
        pltpu.VMEM((_CAP,), jnp.int32),
        pltpu.VMEM((_RCAP + _L, 128), jnp.float32),
        pltpu.VMEM((_RCAP + _L,), jnp.int32),
        pltpu.VMEM((2048,), jnp.int32),
        pltpu.SemaphoreType.DMA,
        pltpu.SemaphoreType.DMA,
    ]

    @functools.partial(
        pl.kernel,
        mesh=mesh,
        out_type=tuple(
            jax.ShapeDtypeStruct((B, 128), jnp.float32) for _ in dims
        ),
        scratch_types=scratch,
        compiler_params=pltpu.CompilerParams(needs_layout_passes=False),
    )
    def gather3(t0, t1, t2, i0, i1, i2, o0, o1, o2,
                win0, win1, lanes, dests, rows, rdst, ivb, s0, s1):
        wid = lax.axis_index("s") * NC + lax.axis_index("c")
        iota = lax.iota(jnp.int32, _L)
        wsem = (s0, s1)

        klo = nwin * wid // NW
        khi = nwin * (wid + 1) // NW

        def wstart(k):
            return jnp.minimum(k * _W, padv - _W)

        for n, (t, i, o, d) in enumerate(
            zip((t0, t1, t2), (i0, i1, i2), (o0, o1, o2), dims)
        ):
            nb = d // 8
            llo = wstart(klo)
            lhi = jnp.where(jnp.int32(khi) >= nwin, jnp.int32(V), khi * _W)

            def chunk_a(c8, n_hits, i=i):
                pltpu.sync_copy(i.at[pl.ds(c8 * 2048, 2048)], ivb)

                def grp_a(g, n_hits):
                    ivec = ivb[pl.ds(g * _L, _L)]
                    m = (ivec >= llo) & (ivec < lhi)
                    mi = m.astype(jnp.int32)
                    pos = n_hits + plsc.cumsum(mi) - mi
                    m = m & (pos < _CAP)
                    plsc.store_scatter(lanes, [pos], ivec, mask=m)
                    plsc.store_scatter(
                        dests, [pos], iota + (c8 * 2048 + g * _L), mask=m)
                    return n_hits + plsc.all_reduce_population_count(m)[0]

                return lax.fori_loop(0, 2048 // _L, grp_a, n_hits)

            n_hits = lax.fori_loop(0, B // 2048, chunk_a, jnp.int32(0))
            ng = (n_hits + _L - 1) // _L

            def rinit(q, _):
                rdst[pl.ds(q * _L, _L)] = jnp.full((_L,), -1, jnp.int32)
                return _
            lax.fori_loop(0, (_RCAP + _L) // _L, rinit, 0)

            pltpu.async_copy(
                t.at[:, :, pl.ds(wstart(klo), _W)],
                win0.at[pl.ds(0, nb)], wsem[0])

            def window_body(p, k, nl, t=t, o=o, d=d, nb=nb):
                wb = (win0, win1)[p]
                wn = (win0, win1)[p ^ 1]

                @pl.when(k + 1 < khi)
                def _():
                    pltpu.async_copy(
                        t.at[:, :, pl.ds(wstart(k + 1), _W)],
                        wn.at[pl.ds(0, nb)], wsem[p ^ 1])

                pltpu.make_async_copy(
                    t.at[:, :, pl.ds(0, _W)],
                    wb.at[pl.ds(0, nb)], wsem[p]).wait()
                ws = wstart(k)

                def grp(g, nl, d=d):
                    lvec = lanes[pl.ds(g * _L, _L)]
                    dvec = dests[pl.ds(g * _L, _L)]
                    m = ((lvec >= ws) & (lvec < ws + _W)
                         & (iota + g * _L < n_hits))
                    cnt = plsc.all_reduce_population_count(m)[0]

                    def do_extract():
                        mi = m.astype(jnp.int32)
                        pos = nl + plsc.cumsum(mi) - mi
                        ll = jnp.where(m, lvec - ws, 0)
                        for c in range(d):
                            av = jnp.full((_L,), c // 8, jnp.int32)
                            sv = jnp.full((_L,), c % 8, jnp.int32)
                            vals = plsc.load_gather(
                                wb, [av, sv, ll], mask=m)
                            plsc.store_scatter(
                                rows, [pos, jnp.full((_L,), c, jnp.int32)],
                                vals, mask=m)
                        plsc.store_scatter(rdst, [pos], dvec, mask=m)
                        return nl + cnt

                    nl = lax.cond(cnt > 0, do_extract, lambda: nl)

                    def do_flush():
                        pltpu.sync_copy(
                            rows,
                            o.at[plsc.Indices(rdst, ignored_value=-1)])
                        return jnp.int32(0)

                    return lax.cond(nl >= _RCAP, do_flush, lambda: nl)

                return lax.fori_loop(0, ng, grp, nl)

            def window(k, nl):
                return lax.cond(
                    (k - klo) % 2 == 0,
                    lambda: window_body(0, k, nl),
                    lambda: window_body(1, k, nl))

            nl = lax.fori_loop(klo, khi, window, jnp.int32(0))

            @pl.when(nl > 0)
            def _():
                pltpu.sync_copy(
                    rows, o.at[plsc.Indices(rdst, ignored_value=-1)])

    o = gather3(*tT3, indices_0, indices_1, indices_2)
    return tuple(oo[:, :d] for oo, d in zip(o, dims))
